# Initial kernel scaffold; baseline (speedup 1.0000x reference)
import jax
import jax.numpy as jnp
from jax.experimental import pallas as pl


def kernel(X, edge_index):
    raise NotImplementedError("write your pallas kernel here")


# speedup vs baseline: 4.2707x; 4.2707x over previous
import functools

import jax
import jax.numpy as jnp
from jax import lax
from jax.experimental import pallas as pl
from jax.experimental.pallas import tpu as pltpu
from jax.experimental.pallas import tpu_sc as plsc

N_NODES = 10000
D_FEAT = 128
N_EDGES = 320000

NC = 2
NS = 16
NW = NC * NS

CHUNK = 128
ACC_ROWS = 10112
ROWS_PER_TILE = ACC_ROWS // NS

T_CHUNKS = -(-N_EDGES // (NW * CHUNK))
E_PAD = NW * CHUNK * T_CHUNKS


def _sc_partial_sums(x, src, dst, zeros):
  mesh = plsc.VectorSubcoreMesh(core_axis_name="c", subcore_axis_name="s")

  @functools.partial(
      pl.kernel,
      mesh=mesh,
      out_type=jax.ShapeDtypeStruct((NC, ACC_ROWS, D_FEAT), jnp.float32),
      scratch_types=[
          pltpu.VMEM((CHUNK,), jnp.int32),
          pltpu.VMEM((CHUNK,), jnp.int32),
          pltpu.VMEM((CHUNK, D_FEAT), jnp.float32),
          pltpu.VMEM_SHARED((ACC_ROWS, D_FEAT), jnp.float32),
          pltpu.SemaphoreType.DMA,
      ],
  )
  def k(x_hbm, src_hbm, dst_hbm, zeros_hbm, out_hbm,
        src_idx_v, dst_idx_v, rows_v, acc, sem):
    c = lax.axis_index("c")
    s = lax.axis_index("s")
    wid = s * NC + c

    pltpu.sync_copy(zeros_hbm, acc.at[pl.ds(s * ROWS_PER_TILE, ROWS_PER_TILE)])
    plsc.subcore_barrier()

    def body(t, carry):
      base = (wid * T_CHUNKS + t) * CHUNK
      pltpu.sync_copy(src_hbm.at[pl.ds(base, CHUNK)], src_idx_v)
      pltpu.sync_copy(dst_hbm.at[pl.ds(base, CHUNK)], dst_idx_v)
      pltpu.async_copy(x_hbm.at[src_idx_v], rows_v, sem).wait()
      pltpu.sync_copy(rows_v, acc.at[dst_idx_v], add=True)
      return carry

    lax.fori_loop(0, T_CHUNKS, body, 0)
    plsc.subcore_barrier()

    pltpu.sync_copy(
        acc.at[pl.ds(s * ROWS_PER_TILE, ROWS_PER_TILE)],
        out_hbm.at[c, pl.ds(s * ROWS_PER_TILE, ROWS_PER_TILE)],
    )

  return k(x, src, dst, zeros)


def _combine_body(a_ref, b_ref, o_ref):
  o_ref[...] = a_ref[0] + b_ref[0]


_BLK = 1000


def _combine(partials):
  return pl.pallas_call(
      _combine_body,
      grid=(N_NODES // _BLK,),
      in_specs=[
          pl.BlockSpec((1, _BLK, D_FEAT), lambda i: (0, i, 0)),
          pl.BlockSpec((1, _BLK, D_FEAT), lambda i: (1, i, 0)),
      ],
      out_specs=pl.BlockSpec((_BLK, D_FEAT), lambda i: (i, 0)),
      out_shape=jax.ShapeDtypeStruct((N_NODES, D_FEAT), jnp.float32),
  )(partials, partials)


def kernel(X, edge_index):
  pad = E_PAD - N_EDGES
  src = jnp.concatenate(
      [edge_index[1], jnp.zeros((pad,), jnp.int32)])
  dst = jnp.concatenate(
      [edge_index[0], jnp.full((pad,), N_NODES, jnp.int32)])
  zeros = jnp.zeros((ROWS_PER_TILE, D_FEAT), jnp.float32)
  partials = _sc_partial_sums(X, src, dst, zeros)
  return _combine(partials)
